# baseline (speedup 1.0000x reference)
import functools

import jax
import jax.numpy as jnp
from jax.experimental import pallas as pl
from jax.experimental.pallas import tpu as pltpu

N = 10000
D = 256
H = 64
C = 3
CP = 8
RB = 200
CBW = 512


def _xw_kernel(x_ref, w1_ref, o_ref):
    o_ref[...] = jnp.dot(x_ref[...], w1_ref[...],
                         preferred_element_type=jnp.float32)


def _pass1_kernel(a_ref, xw_ref, b1_ref, w2_ref, hw_ref, part_ref, hwv_ref):
    i = pl.program_id(0)

    slab = a_ref[...]
    h = jnp.maximum(
        jnp.dot(slab, xw_ref[...], preferred_element_type=jnp.float32)
        + b1_ref[...], 0.0)
    hw_i = jnp.dot(h, w2_ref[...],
                   preferred_element_type=jnp.float32).astype(jnp.bfloat16)

    @pl.when(i == 0)
    def _():
        hwv_ref[...] = jnp.zeros_like(hwv_ref)

    hwv_ref[pl.ds(i * RB, RB), :] = hw_i
    part_ref[...] = jnp.dot(slab.astype(jnp.bfloat16), hwv_ref[...],
                            preferred_element_type=jnp.float32)
    hw_ref[...] = hw_i


def _pass2_kernel(n, a_ref, hw_ref, part_ref, b2_ref, out_ref, acc_ref):
    i = pl.program_id(0)
    j = pl.program_id(1)
    ncb = pl.num_programs(1)
    boundary = (i + 1) * RB
    jstart = boundary // CBW
    jc = jnp.maximum(j, jstart)

    @pl.when(j == 0)
    def _():
        acc_ref[...] = part_ref[...]

    @pl.when(j >= jstart)
    def _():
        col = jc * CBW + jax.lax.broadcasted_iota(jnp.int32, (RB, CBW), 1)
        a_blk = jnp.where((col >= boundary) & (col < n), a_ref[...],
                          0.0).astype(jnp.bfloat16)
        row = jc * CBW + jax.lax.broadcasted_iota(jnp.int32, (CBW, CP), 0)
        hw_blk = jnp.where(row < n, hw_ref[pl.ds(jc * CBW, CBW), :],
                           jnp.bfloat16(0))
        acc_ref[...] += jnp.dot(a_blk, hw_blk,
                                preferred_element_type=jnp.float32)

    @pl.when(j == ncb - 1)
    def _():
        logits = acc_ref[...] + b2_ref[...]
        lane = jax.lax.broadcasted_iota(jnp.int32, logits.shape, 1)
        logits = jnp.where(lane < C, logits, -1e30)
        m = jnp.max(logits, axis=-1, keepdims=True)
        e = jnp.exp(logits - m)
        s = jnp.sum(e, axis=-1, keepdims=True)
        out_ref[...] = (e / s)[:, :C]


@jax.jit
def kernel(x, a, W1, b1, W2, b2):
    n = a.shape[0]
    nr = n // RB
    ncb = -(-n // CBW)
    npad = ncb * CBW

    xw = pl.pallas_call(
        _xw_kernel,
        grid=(n // 1000,),
        in_specs=[
            pl.BlockSpec((1000, D), lambda i: (i, 0)),
            pl.BlockSpec((D, H), lambda i: (0, 0)),
        ],
        out_specs=pl.BlockSpec((1000, H), lambda i: (i, 0)),
        out_shape=jax.ShapeDtypeStruct((n, H), jnp.float32),
    )(x, W1)

    w2p = jnp.zeros((H, CP), jnp.float32).at[:, :C].set(W2)
    b1r = b1.reshape(1, H)
    b2p = jnp.zeros((1, CP), jnp.float32).at[0, :C].set(b2)

    hw, part = pl.pallas_call(
        _pass1_kernel,
        grid=(nr,),
        in_specs=[
            pl.BlockSpec((RB, n), lambda i: (i, 0)),
            pl.BlockSpec((n, H), lambda i: (0, 0)),
            pl.BlockSpec((1, H), lambda i: (0, 0)),
            pl.BlockSpec((H, CP), lambda i: (0, 0)),
        ],
        out_specs=[
            pl.BlockSpec((RB, CP), lambda i: (i, 0)),
            pl.BlockSpec((RB, CP), lambda i: (i, 0)),
        ],
        out_shape=[
            jax.ShapeDtypeStruct((npad, CP), jnp.bfloat16),
            jax.ShapeDtypeStruct((n, CP), jnp.float32),
        ],
        scratch_shapes=[pltpu.VMEM((n, CP), jnp.bfloat16)],
        compiler_params=pltpu.CompilerParams(
            dimension_semantics=("arbitrary",)),
    )(a, xw, b1r, w2p)

    out = pl.pallas_call(
        functools.partial(_pass2_kernel, n),
        grid=(nr, ncb),
        in_specs=[
            pl.BlockSpec(
                (RB, CBW),
                lambda i, j: (i, jnp.maximum(j, ((i + 1) * RB) // CBW))),
            pl.BlockSpec((npad, CP), lambda i, j: (0, 0)),
            pl.BlockSpec((RB, CP), lambda i, j: (i, 0)),
            pl.BlockSpec((1, CP), lambda i, j: (0, 0)),
        ],
        out_specs=pl.BlockSpec((RB, C), lambda i, j: (i, 0)),
        out_shape=jax.ShapeDtypeStruct((n, C), jnp.float32),
        scratch_shapes=[pltpu.VMEM((RB, CP), jnp.float32)],
        compiler_params=pltpu.CompilerParams(
            dimension_semantics=("parallel", "arbitrary")),
    )(a, hw, part, b2p)

    return out


# speedup vs baseline: 2.5342x; 2.5342x over previous
import functools

import jax
import jax.numpy as jnp
from jax.experimental import pallas as pl
from jax.experimental.pallas import tpu as pltpu

N = 10000
D = 256
H = 64
C = 3
CP = 8
RB = 400


def _xw_kernel(x_ref, w1_ref, o_ref):
    o_ref[...] = jnp.dot(x_ref[...], w1_ref[...],
                         preferred_element_type=jnp.float32
                         ).astype(jnp.bfloat16)


def _layer1_kernel(a_ref, xw_ref, b1_ref, w2_ref, hw_ref):
    a_bf = a_ref[...].astype(jnp.bfloat16)
    acc = jnp.dot(a_bf, xw_ref[...], preferred_element_type=jnp.float32)
    h = jnp.maximum(acc + b1_ref[...], 0.0)
    hw_ref[...] = jnp.dot(h, w2_ref[...],
                          preferred_element_type=jnp.float32
                          ).astype(jnp.bfloat16)


def _layer2_kernel(a_ref, hw_ref, b2_ref, out_ref):
    a_bf = a_ref[...].astype(jnp.bfloat16)
    logits = jnp.dot(a_bf, hw_ref[...],
                     preferred_element_type=jnp.float32) + b2_ref[...]
    lane = jax.lax.broadcasted_iota(jnp.int32, logits.shape, 1)
    logits = jnp.where(lane < C, logits, -1e30)
    m = jnp.max(logits, axis=-1, keepdims=True)
    e = jnp.exp(logits - m)
    s = jnp.sum(e, axis=-1, keepdims=True)
    out_ref[...] = (e / s)[:, :C]


@jax.jit
def kernel(x, a, W1, b1, W2, b2):
    n = a.shape[0]
    nr = n // RB

    xw = pl.pallas_call(
        _xw_kernel,
        grid=(n // 1000,),
        in_specs=[
            pl.BlockSpec((1000, D), lambda i: (i, 0)),
            pl.BlockSpec((D, H), lambda i: (0, 0)),
        ],
        out_specs=pl.BlockSpec((1000, H), lambda i: (i, 0)),
        out_shape=jax.ShapeDtypeStruct((n, H), jnp.bfloat16),
    )(x, W1)

    w2p = jnp.zeros((H, CP), jnp.float32).at[:, :C].set(W2)
    b1r = b1.reshape(1, H)
    b2p = jnp.zeros((1, CP), jnp.float32).at[0, :C].set(b2)

    hw = pl.pallas_call(
        _layer1_kernel,
        grid=(nr,),
        in_specs=[
            pl.BlockSpec((RB, n), lambda i: (i, 0)),
            pl.BlockSpec((n, H), lambda i: (0, 0)),
            pl.BlockSpec((1, H), lambda i: (0, 0)),
            pl.BlockSpec((H, CP), lambda i: (0, 0)),
        ],
        out_specs=pl.BlockSpec((RB, CP), lambda i: (i, 0)),
        out_shape=jax.ShapeDtypeStruct((n, CP), jnp.bfloat16),
        compiler_params=pltpu.CompilerParams(
            dimension_semantics=("arbitrary",)),
    )(a, xw, b1r, w2p)

    out = pl.pallas_call(
        _layer2_kernel,
        grid=(nr,),
        in_specs=[
            pl.BlockSpec((RB, n), lambda i: (i, 0)),
            pl.BlockSpec((n, CP), lambda i: (0, 0)),
            pl.BlockSpec((1, CP), lambda i: (0, 0)),
        ],
        out_specs=pl.BlockSpec((RB, C), lambda i: (i, 0)),
        out_shape=jax.ShapeDtypeStruct((n, C), jnp.float32),
        compiler_params=pltpu.CompilerParams(
            dimension_semantics=("arbitrary",)),
    )(a, hw, b2p)

    return out
